# baseline (speedup 1.0000x reference)
import jax
import jax.numpy as jnp
from jax.experimental import pallas as pl
from jax.experimental.pallas import tpu as pltpu
from jax.experimental.pallas import tpu_sc as plsc

BATCH = 1024
SEQ = 200
DIM = 64
FLAT = BATCH * SEQ
LANES = 16
SPLIT = 128


def _encoder_call(text_flat, embedding_weight, pos):
    mesh = plsc.VectorSubcoreMesh(core_axis_name="c", subcore_axis_name="s")

    @pl.kernel(
        out_type=jax.ShapeDtypeStruct((FLAT, DIM), jnp.float32),
        mesh=mesh,
        scratch_types=[pltpu.VMEM((SEQ, DIM), jnp.float32)],
        compiler_params=pltpu.CompilerParams(use_tc_tiling_on_sc=False),
    )
    def enc_kernel(table_hbm, idx_hbm, pos_hbm, out_hbm, pos_v):
        pltpu.sync_copy(pos_hbm, pos_v)

        def body(i_vmem, o_vmem):
            pltpu.sync_copy(
                table_hbm.at[i_vmem.at[0, pl.ds(0, SPLIT)]],
                o_vmem.at[pl.ds(0, SPLIT), :],
            )
            pltpu.sync_copy(
                table_hbm.at[i_vmem.at[0, pl.ds(SPLIT, SEQ - SPLIT)]],
                o_vmem.at[pl.ds(SPLIT, SEQ - SPLIT), :],
            )

            @pl.loop(0, SEQ)
            def _(r):
                for c in range(0, DIM, LANES):
                    o_vmem[r, pl.ds(c, LANES)] = (
                        o_vmem[r, pl.ds(c, LANES)] + pos_v[r, pl.ds(c, LANES)]
                    )

        pltpu.emit_pipeline(
            body,
            grid=(BATCH,),
            in_specs=[pl.BlockSpec((1, SEQ), lambda b: (b, 0))],
            out_specs=[pl.BlockSpec((SEQ, DIM), lambda b: (b, 0))],
            core_axis_name=("c", "s"),
            dimension_semantics=(pltpu.PARALLEL,),
        )(idx_hbm, out_hbm)


    return enc_kernel(embedding_weight, text_flat, pos)


def kernel(text, embedding_weight, positional_encoding):
    seq_len = text.shape[1]
    text_flat = text.astype(jnp.int32)
    pos = positional_encoding[0, :seq_len, :]
    out = _encoder_call(text_flat, embedding_weight, pos)
    return out.reshape(BATCH, SEQ, DIM)


# speedup vs baseline: 1.1642x; 1.1642x over previous
import jax
import jax.numpy as jnp
from jax.experimental import pallas as pl
from jax.experimental.pallas import tpu as pltpu
from jax.experimental.pallas import tpu_sc as plsc

BATCH = 1024
SEQ = 200
DIM = 64
FLAT = BATCH * SEQ
LANES = 16
SPLIT = 128


def _encoder_call(text_flat, embedding_weight, pos):
    mesh = plsc.VectorSubcoreMesh(core_axis_name="c", subcore_axis_name="s")

    @pl.kernel(
        out_type=jax.ShapeDtypeStruct((FLAT, DIM), jnp.float32),
        mesh=mesh,
        scratch_types=[pltpu.VMEM((SEQ, DIM), jnp.float32)],
        compiler_params=pltpu.CompilerParams(use_tc_tiling_on_sc=False),
    )
    def enc_kernel(table_hbm, idx_hbm, pos_hbm, out_hbm, pos_v):
        pltpu.sync_copy(pos_hbm, pos_v)

        def body(i_vmem, o_vmem):
            pltpu.sync_copy(
                table_hbm.at[i_vmem.at[0, pl.ds(0, SPLIT)]],
                o_vmem.at[pl.ds(0, SPLIT), :],
            )
            pltpu.sync_copy(
                table_hbm.at[i_vmem.at[0, pl.ds(SPLIT, SEQ - SPLIT)]],
                o_vmem.at[pl.ds(SPLIT, SEQ - SPLIT), :],
            )


        pltpu.emit_pipeline(
            body,
            grid=(BATCH,),
            in_specs=[pl.BlockSpec((1, SEQ), lambda b: (b, 0))],
            out_specs=[pl.BlockSpec((SEQ, DIM), lambda b: (b, 0))],
            core_axis_name=("c", "s"),
            dimension_semantics=(pltpu.PARALLEL,),
        )(idx_hbm, out_hbm)


    return enc_kernel(embedding_weight, text_flat, pos)


def kernel(text, embedding_weight, positional_encoding):
    seq_len = text.shape[1]
    text_flat = text.astype(jnp.int32)
    pos = positional_encoding[0, :seq_len, :]
    out = _encoder_call(text_flat, embedding_weight, pos)
    return out.reshape(BATCH, SEQ, DIM)


# speedup vs baseline: 1.1956x; 1.0270x over previous
import jax
import jax.numpy as jnp
from jax import lax
from jax.experimental import pallas as pl
from jax.experimental.pallas import tpu as pltpu
from jax.experimental.pallas import tpu_sc as plsc

BATCH = 1024
SEQ = 200
DIM = 64
FLAT = BATCH * SEQ
LANES = 16
NC = 2
NS = 16
NW = NC * NS
CHUNKS = BATCH // NW
NBUF = 4
SPLIT = 128


def _encoder_call(text_ids, embedding_weight, pos):
    mesh = plsc.VectorSubcoreMesh(core_axis_name="c", subcore_axis_name="s")

    @pl.kernel(
        out_type=jax.ShapeDtypeStruct((FLAT, DIM), jnp.float32),
        mesh=mesh,
        scratch_types=[
            pltpu.VMEM((CHUNKS, SEQ), jnp.int32),
            pltpu.VMEM((SEQ, DIM), jnp.float32),
            pltpu.VMEM((NBUF, SEQ, DIM), jnp.float32),
            pltpu.SemaphoreType.DMA((NBUF,)),
            pltpu.SemaphoreType.DMA((NBUF,)),
        ],
        compiler_params=pltpu.CompilerParams(use_tc_tiling_on_sc=False),
    )
    def enc_kernel(table_hbm, idx_hbm, pos_hbm, out_hbm, idx_v, pos_v, rows_v,
                   gsem, osem):
        wid = lax.axis_index("s") * NC + lax.axis_index("c")
        base_row = wid * CHUNKS
        base_out = base_row * SEQ

        pltpu.sync_copy(idx_hbm.at[pl.ds(base_row, CHUNKS), :], idx_v)
        pltpu.sync_copy(pos_hbm, pos_v)

        def gather_descs(q, b):
            return (
                pltpu.make_async_copy(
                    table_hbm.at[idx_v.at[q, pl.ds(0, SPLIT)]],
                    rows_v.at[b, pl.ds(0, SPLIT), :],
                    gsem.at[b],
                ),
                pltpu.make_async_copy(
                    table_hbm.at[idx_v.at[q, pl.ds(SPLIT, SEQ - SPLIT)]],
                    rows_v.at[b, pl.ds(SPLIT, SEQ - SPLIT), :],
                    gsem.at[b],
                ),
            )

        def out_desc(q, b):
            return pltpu.make_async_copy(
                rows_v.at[b],
                out_hbm.at[pl.ds(base_out + q * SEQ, SEQ), :],
                osem.at[b],
            )

        def start_gather(q, b):
            d1, d2 = gather_descs(q, b)
            d1.start()
            d2.start()

        def wait_gather(q, b):
            d1, d2 = gather_descs(q, b)
            d1.wait()
            d2.wait()

        for j in range(NBUF - 1):
            start_gather(j, j)

        @pl.loop(0, CHUNKS, step=NBUF)
        def _(c0):
            for j in range(NBUF):
                q = c0 + j
                qpre = q + NBUF - 1
                bpre = (j + NBUF - 1) % NBUF

                @pl.when(qpre < CHUNKS)
                def _():
                    @pl.when(q >= 1)
                    def _():
                        out_desc(q - 1, bpre).wait()

                    start_gather(qpre, bpre)

                wait_gather(q, j)

                @pl.loop(0, SEQ, step=2)
                def _(r):
                    for rr in range(2):
                        for cc in range(0, DIM, LANES):
                            plsc.addupdate(
                                rows_v.at[j, r + rr, pl.ds(cc, LANES)],
                                pos_v[r + rr, pl.ds(cc, LANES)],
                            )

                out_desc(q, j).start()

        for j in range(NBUF):
            out_desc(CHUNKS - NBUF + j, j).wait()

    return enc_kernel(embedding_weight, text_ids, pos)


def kernel(text, embedding_weight, positional_encoding):
    seq_len = text.shape[1]
    text_ids = text.astype(jnp.int32)
    pos = positional_encoding[0, :seq_len, :]
    out = _encoder_call(text_ids, embedding_weight, pos)
    return out.reshape(BATCH, SEQ, DIM)
